# Initial kernel scaffold; baseline (speedup 1.0000x reference)
import jax
import jax.numpy as jnp
from jax.experimental import pallas as pl


def kernel(x, position_tags, emb_table, gamma, beta):
    raise NotImplementedError("write your pallas kernel here")


# speedup vs baseline: 3.7061x; 3.7061x over previous
import functools

import jax
import jax.numpy as jnp
from jax.experimental import pallas as pl

_EPS = 1e-12
_VP = 512


def _body(x_ref, idx_ref, tab_ref, g_ref, b_ref, o_ref, *, rows, vp):
    idx = idx_ref[0, 0, :]
    iota = jax.lax.broadcasted_iota(jnp.int32, (rows, vp), 1)
    onehot = (idx[:, None] == iota).astype(jnp.bfloat16)
    pe = jax.lax.dot_general(
        onehot, tab_ref[...],
        dimension_numbers=(((1,), (0,)), ((), ())),
        preferred_element_type=jnp.float32,
    )
    h = x_ref[...] + pe
    mean = jnp.mean(h, axis=1, keepdims=True)
    c = h - mean
    var = jnp.mean(c * c, axis=1, keepdims=True)
    o_ref[...] = (c * jax.lax.rsqrt(var + _EPS)) * g_ref[...] + b_ref[...]


def kernel(x, position_tags, emb_table, gamma, beta):
    b, l, f = x.shape
    n = b * l
    rows = 2048 if n % 2048 == 0 else n
    nb = n // rows
    xf = x.reshape(n, f)
    idx = position_tags.astype(jnp.int32).reshape(nb, 1, rows)
    tab = jnp.pad(
        emb_table, ((0, _VP - emb_table.shape[0]), (0, 0))
    ).astype(jnp.bfloat16)
    g2 = gamma.reshape(1, f)
    b2 = beta.reshape(1, f)
    out = pl.pallas_call(
        functools.partial(_body, rows=rows, vp=_VP),
        grid=(nb,),
        in_specs=[
            pl.BlockSpec((rows, f), lambda i: (i, 0)),
            pl.BlockSpec((1, 1, rows), lambda i: (i, 0, 0)),
            pl.BlockSpec((_VP, f), lambda i: (0, 0)),
            pl.BlockSpec((1, f), lambda i: (0, 0)),
            pl.BlockSpec((1, f), lambda i: (0, 0)),
        ],
        out_specs=pl.BlockSpec((rows, f), lambda i: (i, 0)),
        out_shape=jax.ShapeDtypeStruct((n, f), jnp.float32),
    )(xf, idx, tab, g2, b2)
    return out.reshape(b, l, f)
